# Initial kernel scaffold; baseline (speedup 1.0000x reference)
import functools

import jax
import jax.numpy as jnp
from jax import lax
from jax.experimental import pallas as pl
from jax.experimental.pallas import tpu as pltpu
from jax.experimental.pallas import tpu_sc as plsc

NUM_ROWS = 50000
NUM_COLS = 256
NUM_SEGS = 64
LANES = 16
VREGS_PER_ROW = NUM_COLS // LANES
NUM_CORES = 2
NUM_SUBCORES = 16
NUM_WORKERS = NUM_CORES * NUM_SUBCORES
SEGS_PER_WORKER = NUM_SEGS // NUM_WORKERS
CHUNK_ROWS = 128


def _sc_body(x_hbm, starts_hbm, out_hbm, starts_v, buf, out_v):
    w = lax.axis_index("s") * NUM_CORES + lax.axis_index("c")
    pltpu.sync_copy(starts_hbm, starts_v)

    for si in range(SEGS_PER_WORKER):
        seg = SEGS_PER_WORKER * w + si
        lo = starts_v[seg]
        hi = starts_v[seg + 1]
        n = hi - lo
        nch = (n + CHUNK_ROWS - 1) // CHUNK_ROWS

        def chunk_body(c, accs, lo=lo, n=n):
            start = lo + c * CHUNK_ROWS
            st = jnp.minimum(start, NUM_ROWS - CHUNK_ROWS)
            off = start - st
            cnt = jnp.minimum(CHUNK_ROWS, n - c * CHUNK_ROWS)
            pltpu.sync_copy(x_hbm.at[pl.ds(st, CHUNK_ROWS)], buf)

            def row_body(i, accs, off=off):
                r = off + i
                return tuple(
                    jnp.maximum(accs[d], buf[r, pl.ds(LANES * d, LANES)])
                    for d in range(VREGS_PER_ROW)
                )

            return lax.fori_loop(0, cnt, row_body, accs)

        neg_inf = jnp.full((LANES,), -jnp.inf, dtype=jnp.float32)
        accs = lax.fori_loop(
            0, nch, chunk_body, tuple(neg_inf for _ in range(VREGS_PER_ROW))
        )
        for d in range(VREGS_PER_ROW):
            out_v[si, pl.ds(LANES * d, LANES)] = accs[d]

    pltpu.sync_copy(out_v, out_hbm.at[pl.ds(SEGS_PER_WORKER * w, SEGS_PER_WORKER)])


@jax.jit
def kernel(x, batch):
    batch = batch.astype(jnp.int32)
    queries = jnp.arange(NUM_SEGS + 1, dtype=jnp.int32)
    starts = jnp.searchsorted(
        batch, queries, side="left", method="compare_all"
    ).astype(jnp.int32)
    starts = jnp.full((80,), NUM_ROWS, dtype=jnp.int32).at[: NUM_SEGS + 1].set(starts)

    mesh = plsc.VectorSubcoreMesh(core_axis_name="c", subcore_axis_name="s")
    run = functools.partial(
        pl.kernel,
        mesh=mesh,
        out_type=jax.ShapeDtypeStruct((NUM_SEGS, NUM_COLS), jnp.float32),
        scratch_types=[
            pltpu.VMEM((80,), jnp.int32),
            pltpu.VMEM((CHUNK_ROWS, NUM_COLS), jnp.float32),
            pltpu.VMEM((SEGS_PER_WORKER, NUM_COLS), jnp.float32),
        ],
    )(_sc_body)
    return run(x, starts)


# speedup vs baseline: 6.9749x; 6.9749x over previous
import functools

import jax
import jax.numpy as jnp
from jax import lax
from jax.experimental import pallas as pl
from jax.experimental.pallas import tpu as pltpu
from jax.experimental.pallas import tpu_sc as plsc

NUM_ROWS = 50000
NUM_COLS = 256
NUM_SEGS = 64
LANES = 16
VREGS_PER_ROW = NUM_COLS // LANES
NUM_CORES = 2
NUM_SUBCORES = 16
NUM_WORKERS = NUM_CORES * NUM_SUBCORES
SEGS_PER_WORKER = NUM_SEGS // NUM_WORKERS
CHUNK_ROWS = 128


def _sc_body(x_hbm, starts_hbm, out_hbm, starts_v, buf, out_v):
    w = lax.axis_index("s") * NUM_CORES + lax.axis_index("c")
    pltpu.sync_copy(starts_hbm, starts_v)
    bounds = starts_v[pl.ds(SEGS_PER_WORKER * w, LANES)]

    for si in range(SEGS_PER_WORKER):
        lo = bounds[si]
        hi = bounds[si + 1]
        a = (lo // 8) * 8
        nch = (hi - a + CHUNK_ROWS - 1) // CHUNK_ROWS

        def chunk_body(c, accs, lo=lo, hi=hi, a=a):
            base = jnp.minimum(a + c * CHUNK_ROWS, NUM_ROWS - CHUNK_ROWS)
            base = pl.multiple_of(base, 8)
            pltpu.sync_copy(x_hbm.at[pl.ds(base, CHUNK_ROWS)], buf)
            i0 = jnp.maximum(lo - base, 0)
            i1 = jnp.minimum(hi - base, CHUNK_ROWS)

            def row_body(i, accs):
                return tuple(
                    jnp.maximum(accs[d], buf[i, pl.ds(LANES * d, LANES)])
                    for d in range(VREGS_PER_ROW)
                )

            return lax.fori_loop(i0, i1, row_body, accs)

        neg_inf = jnp.full((LANES,), -jnp.inf, dtype=jnp.float32)
        accs = lax.fori_loop(
            0, nch, chunk_body, tuple(neg_inf for _ in range(VREGS_PER_ROW))
        )
        for d in range(VREGS_PER_ROW):
            out_v[si, pl.ds(LANES * d, LANES)] = accs[d]

    pltpu.sync_copy(out_v, out_hbm.at[pl.ds(SEGS_PER_WORKER * w, SEGS_PER_WORKER)])


@jax.jit
def kernel(x, batch):
    batch = batch.astype(jnp.int32)
    queries = jnp.arange(NUM_SEGS + 1, dtype=jnp.int32)
    starts = jnp.searchsorted(
        batch, queries, side="left", method="compare_all"
    ).astype(jnp.int32)
    starts = jnp.full((80,), NUM_ROWS, dtype=jnp.int32).at[: NUM_SEGS + 1].set(starts)

    mesh = plsc.VectorSubcoreMesh(core_axis_name="c", subcore_axis_name="s")
    run = functools.partial(
        pl.kernel,
        mesh=mesh,
        out_type=jax.ShapeDtypeStruct((NUM_SEGS, NUM_COLS), jnp.float32),
        scratch_types=[
            pltpu.VMEM((80,), jnp.int32),
            pltpu.VMEM((CHUNK_ROWS, NUM_COLS), jnp.float32),
            pltpu.VMEM((SEGS_PER_WORKER, NUM_COLS), jnp.float32),
        ],
    )(_sc_body)
    return run(x, starts)


# speedup vs baseline: 7.0166x; 1.0060x over previous
import functools

import jax
import jax.numpy as jnp
from jax import lax
from jax.experimental import pallas as pl
from jax.experimental.pallas import tpu as pltpu
from jax.experimental.pallas import tpu_sc as plsc

NUM_ROWS = 50000
NUM_COLS = 256
NUM_SEGS = 64
LANES = 16
VREGS_PER_ROW = NUM_COLS // LANES
NUM_CORES = 2
NUM_SUBCORES = 16
NUM_WORKERS = NUM_CORES * NUM_SUBCORES
SEGS_PER_WORKER = NUM_SEGS // NUM_WORKERS
CHUNK_ROWS = 128


def _sc_body(x_hbm, starts_hbm, out_hbm, starts_v, buf, out_v):
    w = lax.axis_index("s") * NUM_CORES + lax.axis_index("c")
    pltpu.sync_copy(starts_hbm, starts_v)
    bounds = starts_v[pl.ds(SEGS_PER_WORKER * w, LANES)]

    for si in range(SEGS_PER_WORKER):
        lo = bounds[si]
        hi = bounds[si + 1]
        a = (lo // 8) * 8
        nch = (hi - a + CHUNK_ROWS - 1) // CHUNK_ROWS

        def chunk_body(c, accs, lo=lo, hi=hi, a=a):
            base = jnp.minimum(a + c * CHUNK_ROWS, NUM_ROWS - CHUNK_ROWS)
            base = pl.multiple_of(base, 8)
            pltpu.sync_copy(x_hbm.at[pl.ds(base, CHUNK_ROWS)], buf)
            i0 = jnp.maximum(lo - base, 0)
            i1 = jnp.minimum(hi - base, CHUNK_ROWS)

            def row_body(i, accs):
                return tuple(
                    jnp.maximum(accs[d], buf[i, pl.ds(LANES * d, LANES)])
                    for d in range(VREGS_PER_ROW)
                )

            return plsc.parallel_loop(i0, i1, 1, unroll=8, carry=accs)(row_body)

        neg_inf = jnp.full((LANES,), -jnp.inf, dtype=jnp.float32)
        accs = lax.fori_loop(
            0, nch, chunk_body, tuple(neg_inf for _ in range(VREGS_PER_ROW))
        )
        for d in range(VREGS_PER_ROW):
            out_v[si, pl.ds(LANES * d, LANES)] = accs[d]

    pltpu.sync_copy(out_v, out_hbm.at[pl.ds(SEGS_PER_WORKER * w, SEGS_PER_WORKER)])


@jax.jit
def kernel(x, batch):
    batch = batch.astype(jnp.int32)
    queries = jnp.arange(NUM_SEGS + 1, dtype=jnp.int32)
    starts = jnp.searchsorted(
        batch, queries, side="left", method="compare_all"
    ).astype(jnp.int32)
    starts = jnp.full((80,), NUM_ROWS, dtype=jnp.int32).at[: NUM_SEGS + 1].set(starts)

    mesh = plsc.VectorSubcoreMesh(core_axis_name="c", subcore_axis_name="s")
    run = functools.partial(
        pl.kernel,
        mesh=mesh,
        out_type=jax.ShapeDtypeStruct((NUM_SEGS, NUM_COLS), jnp.float32),
        scratch_types=[
            pltpu.VMEM((80,), jnp.int32),
            pltpu.VMEM((CHUNK_ROWS, NUM_COLS), jnp.float32),
            pltpu.VMEM((SEGS_PER_WORKER, NUM_COLS), jnp.float32),
        ],
    )(_sc_body)
    return run(x, starts)


# speedup vs baseline: 9.0063x; 1.2836x over previous
import functools

import jax
import jax.numpy as jnp
from jax import lax
from jax.experimental import pallas as pl
from jax.experimental.pallas import tpu as pltpu
from jax.experimental.pallas import tpu_sc as plsc

NUM_ROWS = 50000
NUM_COLS = 256
NUM_SEGS = 64
LANES = 16
VREGS_PER_ROW = NUM_COLS // LANES
NUM_CORES = 2
NUM_SUBCORES = 16
NUM_WORKERS = NUM_CORES * NUM_SUBCORES
SEGS_PER_WORKER = NUM_SEGS // NUM_WORKERS
CHUNK_ROWS = 128


def _sc_body(x_hbm, starts_hbm, out_hbm, starts_v, buf, out_v, sems):
    w = lax.axis_index("s") * NUM_CORES + lax.axis_index("c")
    pltpu.sync_copy(starts_hbm, starts_v)
    bounds = starts_v[pl.ds(SEGS_PER_WORKER * w, LANES)]

    for si in range(SEGS_PER_WORKER):
        lo = bounds[si]
        hi = bounds[si + 1]
        a = (lo // 8) * 8
        nch = jnp.maximum((hi - a + CHUNK_ROWS - 1) // CHUNK_ROWS, 1)

        def base_of(c, a=a):
            return pl.multiple_of(
                jnp.minimum(a + c * CHUNK_ROWS, NUM_ROWS - CHUNK_ROWS), 8
            )

        pltpu.async_copy(x_hbm.at[pl.ds(base_of(0), CHUNK_ROWS)], buf.at[0], sems.at[0])

        def chunk_body(c, accs, lo=lo, hi=hi, nch=nch, base_of=base_of):
            p = c % 2
            base = base_of(c)
            pltpu.make_async_copy(
                x_hbm.at[pl.ds(base, CHUNK_ROWS)], buf.at[p], sems.at[p]
            ).wait()

            @pl.when(c + 1 < nch)
            def _():
                pltpu.async_copy(
                    x_hbm.at[pl.ds(base_of(c + 1), CHUNK_ROWS)],
                    buf.at[1 - p],
                    sems.at[1 - p],
                )

            i0 = jnp.maximum(lo - base, 0)
            i1 = jnp.minimum(hi - base, CHUNK_ROWS)

            def row_body(i, accs):
                return tuple(
                    jnp.maximum(accs[d], buf[p, i, pl.ds(LANES * d, LANES)])
                    for d in range(VREGS_PER_ROW)
                )

            return plsc.parallel_loop(i0, i1, 1, unroll=8, carry=accs)(row_body)

        neg_inf = jnp.full((LANES,), -jnp.inf, dtype=jnp.float32)
        accs = lax.fori_loop(
            0, nch, chunk_body, tuple(neg_inf for _ in range(VREGS_PER_ROW))
        )
        for d in range(VREGS_PER_ROW):
            out_v[si, pl.ds(LANES * d, LANES)] = accs[d]

    pltpu.sync_copy(out_v, out_hbm.at[pl.ds(SEGS_PER_WORKER * w, SEGS_PER_WORKER)])


@jax.jit
def kernel(x, batch):
    batch = batch.astype(jnp.int32)
    queries = jnp.arange(NUM_SEGS + 1, dtype=jnp.int32)
    starts = jnp.searchsorted(
        batch, queries, side="left", method="compare_all"
    ).astype(jnp.int32)
    starts = jnp.full((80,), NUM_ROWS, dtype=jnp.int32).at[: NUM_SEGS + 1].set(starts)

    mesh = plsc.VectorSubcoreMesh(core_axis_name="c", subcore_axis_name="s")
    run = functools.partial(
        pl.kernel,
        mesh=mesh,
        out_type=jax.ShapeDtypeStruct((NUM_SEGS, NUM_COLS), jnp.float32),
        scratch_types=[
            pltpu.VMEM((80,), jnp.int32),
            pltpu.VMEM((2, CHUNK_ROWS, NUM_COLS), jnp.float32),
            pltpu.VMEM((SEGS_PER_WORKER, NUM_COLS), jnp.float32),
            pltpu.SemaphoreType.DMA((2,)),
        ],
    )(_sc_body)
    return run(x, starts)


# speedup vs baseline: 10.3430x; 1.1484x over previous
import functools

import jax
import jax.numpy as jnp
from jax import lax
from jax.experimental import pallas as pl
from jax.experimental.pallas import tpu as pltpu
from jax.experimental.pallas import tpu_sc as plsc

NUM_ROWS = 50000
NUM_COLS = 256
NUM_SEGS = 64
LANES = 16
VREGS_PER_ROW = NUM_COLS // LANES
NUM_CORES = 2
NUM_SUBCORES = 16
NUM_WORKERS = NUM_CORES * NUM_SUBCORES
SEGS_PER_WORKER = NUM_SEGS // NUM_WORKERS
CHUNK_ROWS = 64
NBUF = 4


def _sc_body(x_hbm, starts_hbm, out_hbm, starts_v, buf, out_v, sems):
    w = lax.axis_index("s") * NUM_CORES + lax.axis_index("c")
    pltpu.sync_copy(starts_hbm, starts_v)
    bounds = starts_v[pl.ds(SEGS_PER_WORKER * w, LANES)]

    for si in range(SEGS_PER_WORKER):
        lo = bounds[si]
        hi = bounds[si + 1]
        a = (lo // 8) * 8
        nch = jnp.maximum((hi - a + CHUNK_ROWS - 1) // CHUNK_ROWS, 1)

        def base_of(c, a=a):
            return pl.multiple_of(
                jnp.minimum(a + c * CHUNK_ROWS, NUM_ROWS - CHUNK_ROWS), 8
            )

        for k in range(NBUF - 1):

            @pl.when(k < nch)
            def _(k=k):
                pltpu.async_copy(
                    x_hbm.at[pl.ds(base_of(k), CHUNK_ROWS)], buf.at[k], sems.at[k]
                )

        def chunk_body(c, accs, lo=lo, hi=hi, nch=nch, base_of=base_of):
            p = c % NBUF
            base = base_of(c)
            pltpu.make_async_copy(
                x_hbm.at[pl.ds(base, CHUNK_ROWS)], buf.at[p], sems.at[p]
            ).wait()

            @pl.when(c + NBUF - 1 < nch)
            def _():
                nxt = (c + NBUF - 1) % NBUF
                pltpu.async_copy(
                    x_hbm.at[pl.ds(base_of(c + NBUF - 1), CHUNK_ROWS)],
                    buf.at[nxt],
                    sems.at[nxt],
                )

            i0 = jnp.maximum(lo - base, 0)
            i1 = jnp.minimum(hi - base, CHUNK_ROWS)

            def row_body(i, accs):
                return tuple(
                    jnp.maximum(accs[d], buf[p, i, pl.ds(LANES * d, LANES)])
                    for d in range(VREGS_PER_ROW)
                )

            return plsc.parallel_loop(i0, i1, 1, unroll=8, carry=accs)(row_body)

        neg_inf = jnp.full((LANES,), -jnp.inf, dtype=jnp.float32)
        accs = lax.fori_loop(
            0, nch, chunk_body, tuple(neg_inf for _ in range(VREGS_PER_ROW))
        )
        for d in range(VREGS_PER_ROW):
            out_v[si, pl.ds(LANES * d, LANES)] = accs[d]

    pltpu.sync_copy(out_v, out_hbm.at[pl.ds(SEGS_PER_WORKER * w, SEGS_PER_WORKER)])


@jax.jit
def kernel(x, batch):
    batch = batch.astype(jnp.int32)
    queries = jnp.arange(NUM_SEGS + 1, dtype=jnp.int32)
    starts = jnp.searchsorted(
        batch, queries, side="left", method="compare_all"
    ).astype(jnp.int32)
    starts = jnp.full((80,), NUM_ROWS, dtype=jnp.int32).at[: NUM_SEGS + 1].set(starts)

    mesh = plsc.VectorSubcoreMesh(core_axis_name="c", subcore_axis_name="s")
    run = functools.partial(
        pl.kernel,
        mesh=mesh,
        out_type=jax.ShapeDtypeStruct((NUM_SEGS, NUM_COLS), jnp.float32),
        scratch_types=[
            pltpu.VMEM((80,), jnp.int32),
            pltpu.VMEM((NBUF, CHUNK_ROWS, NUM_COLS), jnp.float32),
            pltpu.VMEM((SEGS_PER_WORKER, NUM_COLS), jnp.float32),
            pltpu.SemaphoreType.DMA((NBUF,)),
        ],
    )(_sc_body)
    return run(x, starts)
